# baseline (speedup 1.0000x reference)
import functools

import jax
import jax.numpy as jnp
from jax import lax
from jax.experimental import pallas as pl
from jax.experimental.pallas import tpu as pltpu
from jax.experimental.pallas import tpu_sc as plsc

B, C, H, W = 2, 96, 512, 512
HW = H * W
N = B * HW
CP = 128

NC, NS, L = 2, 16, 16
NW = NC * NS
PIX_PER_W = N // NW
K = 128
CHUNKS = PIX_PER_W // K

_HB = 128
_SKIP_COMPUTE = False
_SKIP_GATHER = False

_BCAST_DNUMS = lax.GatherDimensionNumbers(
    offset_dims=(), collapsed_slice_dims=(0,), start_index_map=(0,))


def _lane_bcast(vec, j):
    idx = jnp.full((L, 1), j, jnp.int32)
    return lax.gather(vec, idx, _BCAST_DNUMS, (1,),
                      mode=lax.GatherScatterMode.PROMISE_IN_BOUNDS)


def _prep_body(in_ref, idx_ref, w_ref):
    b = pl.program_id(0)
    h = pl.program_id(1)
    d = in_ref[0]
    dx = d[0]
    dy = d[1]
    gy = lax.broadcasted_iota(jnp.int32, (_HB, W), 0).astype(jnp.float32) + (
        h * _HB).astype(jnp.float32)
    gx = lax.broadcasted_iota(jnp.int32, (_HB, W), 1).astype(jnp.float32)
    xf = gx + dx
    yf = gy + dy
    x0 = jnp.floor(xf)
    y0 = jnp.floor(yf)
    y1 = y0 + 1.0
    wx1 = xf - x0
    wx0 = 1.0 - wx1
    wy1 = yf - y0
    wy0 = 1.0 - wy1
    fW = jnp.float32(W - 1)
    fH = jnp.float32(H - 1)
    xbf = jnp.clip(x0, 0.0, jnp.float32(W - 2))
    eqA = (x0 == xbf).astype(jnp.float32)
    wA = wx0 * eqA + wx1 * (x0 == -1.0).astype(jnp.float32)
    wB = wx1 * eqA + wx0 * (x0 == fW).astype(jnp.float32)
    vy0 = ((y0 >= 0) & (y0 <= fH)).astype(jnp.float32)
    vy1 = ((y1 >= 0) & (y1 <= fH)).astype(jnp.float32)
    xb = xbf.astype(jnp.int32)
    y0c = jnp.clip(y0, 0.0, fH).astype(jnp.int32)
    y1c = jnp.clip(y1, 0.0, fH).astype(jnp.int32)
    base = b * HW
    idx_ref[0, 0] = base + y0c * W + xb
    idx_ref[1, 0] = base + y1c * W + xb
    wT = wy0 * vy0
    wBo = wy1 * vy1
    w_ref[0, 0] = wA * wT
    w_ref[1, 0] = wB * wT
    w_ref[2, 0] = wA * wBo
    w_ref[3, 0] = wB * wBo


def _prep(input2, interpret=False):
    return pl.pallas_call(
        _prep_body,
        grid=(B, H // _HB),
        in_specs=[pl.BlockSpec((1, 2, _HB, W), lambda b, h: (b, 0, h, 0))],
        out_specs=[
            pl.BlockSpec((2, 1, _HB, W), lambda b, h: (0, b, h, 0)),
            pl.BlockSpec((4, 1, _HB, W), lambda b, h: (0, b, h, 0)),
        ],
        out_shape=[
            jax.ShapeDtypeStruct((2, B, H, W), jnp.int32),
            jax.ShapeDtypeStruct((4, B, H, W), jnp.float32),
        ],
        interpret=interpret,
    )(input2)


_M0 = -65536


def _halves(u32):
    ev = plsc.bitcast(u32 << 16, jnp.float32)
    od = plsc.bitcast(u32 & _M0, jnp.float32)
    return ev, od


def _sc_warp_body(table, idx2, w4, out, idx_v, w_v, rows_v, ob_v, sem):
    wid = lax.axis_index("s") * NC + lax.axis_index("c")
    pix0 = wid * PIX_PER_W

    def chunk(g, _):
        base = pix0 + g * K
        pltpu.sync_copy(idx2.at[:, pl.ds(base, K)], idx_v)
        pltpu.sync_copy(w4.at[:, pl.ds(base, K)], w_v)
        if not _SKIP_GATHER:
            descs = [
                pltpu.async_copy(table.at[idx_v.at[i]], rows_v.at[i], sem)
                for i in range(2)
            ]
            for d in descs:
                d.wait()

        def xg_body(xg, _):
            ci = lax.broadcasted_iota(jnp.int32, (L,), 0)
            zz = jnp.full((L,), 0, jnp.int32)
            w16 = [w_v[i, pl.ds(xg * L, L)] for i in range(4)]
            pbase = xg * L
            for j in range(L):
                wj = [_lane_bcast(w16[i], j) for i in range(4)]
                p = pbase + j
                pp = zz + p
                us = [[rows_v[i, p, pl.ds(s * 64 + cb * L, L)]
                       for cb in range(3)]
                      for i in range(2) for s in range(2)]
                for cb in range(3):
                    e0, o0 = _halves(us[0][cb])
                    e1, o1 = _halves(us[1][cb])
                    e2, o2 = _halves(us[2][cb])
                    e3, o3 = _halves(us[3][cb])
                    acc_e = e0 * wj[0] + e1 * wj[1] + e2 * wj[2] + e3 * wj[3]
                    acc_o = o0 * wj[0] + o1 * wj[1] + o2 * wj[2] + o3 * wj[3]
                    plsc.store_scatter(ob_v, [ci + cb * L, pp], acc_e)
                    plsc.store_scatter(ob_v, [ci + (C // 2 + cb * L), pp],
                                       acc_o)
            return 0

        if not _SKIP_COMPUTE:
            lax.fori_loop(0, K // L, xg_body, 0)

        bb = base // HW
        rem = base - bb * HW
        yy = rem // W
        xx = rem - yy * W
        pltpu.sync_copy(ob_v.at[:, pl.ds(0, K)], out.at[bb, :, yy, pl.ds(xx, K)])
        return 0

    lax.fori_loop(0, CHUNKS, chunk, 0)


@functools.lru_cache(maxsize=1)
def _sc_warp():
    return pl.kernel(
        _sc_warp_body,
        out_type=jax.ShapeDtypeStruct((B, C, H, W), jnp.float32),
        mesh=plsc.VectorSubcoreMesh(core_axis_name="c", subcore_axis_name="s"),
        compiler_params=pltpu.CompilerParams(needs_layout_passes=False),
        scratch_types=[
            pltpu.VMEM((2, K), jnp.int32),
            pltpu.VMEM((4, K), jnp.float32),
            pltpu.VMEM((2, K, CP), jnp.int32),
            pltpu.VMEM((C, K + 1), jnp.float32),
            pltpu.SemaphoreType.DMA,
        ],
    )


def _build_table(input1):
    t = jnp.transpose(input1, (0, 2, 3, 1)).reshape(N, C)
    t = t.astype(jnp.bfloat16)
    t = jnp.stack([t[:, : C // 2], t[:, C // 2:]], axis=2)
    ti = lax.bitcast_convert_type(t, jnp.int32)
    ti = jnp.pad(ti, ((0, 0), (0, CP // 2 - C // 2)))
    nxt = jnp.roll(ti, -1, axis=0)
    return jnp.concatenate([ti, nxt], axis=1)


def kernel(input1, input2):
    if input2.shape[1] == 3:
        input2 = input2[:, :2, :, :]
    table = _build_table(input1)
    idx2, w4 = _prep(input2)
    idx2 = idx2.reshape(2, N)
    w4 = w4.reshape(4, N)
    return _sc_warp()(table, idx2, w4)


# speedup vs baseline: 1.0002x; 1.0002x over previous
import functools

import jax
import jax.numpy as jnp
from jax import lax
from jax.experimental import pallas as pl
from jax.experimental.pallas import tpu as pltpu
from jax.experimental.pallas import tpu_sc as plsc

B, C, H, W = 2, 96, 512, 512
HW = H * W
N = B * HW
CP = 128

NC, NS, L = 2, 16, 16
NW = NC * NS
PIX_PER_W = N // NW
K = 128
CHUNKS = PIX_PER_W // K

_HB = 128
_SKIP_COMPUTE = False
_SKIP_GATHER = False

_BCAST_DNUMS = lax.GatherDimensionNumbers(
    offset_dims=(), collapsed_slice_dims=(0,), start_index_map=(0,))


def _lane_bcast(vec, j):
    idx = jnp.full((L, 1), j, jnp.int32)
    return lax.gather(vec, idx, _BCAST_DNUMS, (1,),
                      mode=lax.GatherScatterMode.PROMISE_IN_BOUNDS)


def _prep_body(in_ref, idx_ref, w_ref):
    b = pl.program_id(0)
    h = pl.program_id(1)
    d = in_ref[0]
    dx = d[0]
    dy = d[1]
    gy = lax.broadcasted_iota(jnp.int32, (_HB, W), 0).astype(jnp.float32) + (
        h * _HB).astype(jnp.float32)
    gx = lax.broadcasted_iota(jnp.int32, (_HB, W), 1).astype(jnp.float32)
    xf = gx + dx
    yf = gy + dy
    x0 = jnp.floor(xf)
    y0 = jnp.floor(yf)
    y1 = y0 + 1.0
    wx1 = xf - x0
    wx0 = 1.0 - wx1
    wy1 = yf - y0
    wy0 = 1.0 - wy1
    fW = jnp.float32(W - 1)
    fH = jnp.float32(H - 1)
    xbf = jnp.clip(x0, 0.0, jnp.float32(W - 2))
    eqA = (x0 == xbf).astype(jnp.float32)
    wA = wx0 * eqA + wx1 * (x0 == -1.0).astype(jnp.float32)
    wB = wx1 * eqA + wx0 * (x0 == fW).astype(jnp.float32)
    vy0 = ((y0 >= 0) & (y0 <= fH)).astype(jnp.float32)
    vy1 = ((y1 >= 0) & (y1 <= fH)).astype(jnp.float32)
    xb = xbf.astype(jnp.int32)
    y0c = jnp.clip(y0, 0.0, fH).astype(jnp.int32)
    y1c = jnp.clip(y1, 0.0, fH).astype(jnp.int32)
    base = b * HW
    idx_ref[0, 0] = base + y0c * W + xb
    idx_ref[1, 0] = base + y1c * W + xb
    wT = wy0 * vy0
    wBo = wy1 * vy1
    w_ref[0, 0] = wA * wT
    w_ref[1, 0] = wB * wT
    w_ref[2, 0] = wA * wBo
    w_ref[3, 0] = wB * wBo


def _prep(input2, interpret=False):
    return pl.pallas_call(
        _prep_body,
        grid=(B, H // _HB),
        in_specs=[pl.BlockSpec((1, 2, _HB, W), lambda b, h: (b, 0, h, 0))],
        out_specs=[
            pl.BlockSpec((2, 1, _HB, W), lambda b, h: (0, b, h, 0)),
            pl.BlockSpec((4, 1, _HB, W), lambda b, h: (0, b, h, 0)),
        ],
        out_shape=[
            jax.ShapeDtypeStruct((2, B, H, W), jnp.int32),
            jax.ShapeDtypeStruct((4, B, H, W), jnp.float32),
        ],
        interpret=interpret,
    )(input2)


_M0 = -65536


def _halves(u32):
    ev = plsc.bitcast(u32 << 16, jnp.float32)
    od = plsc.bitcast(u32 & _M0, jnp.float32)
    return ev, od


def _sc_warp_body(table, idx2, w4, out, idx_v, w_v, rows_v, ob_v, sem):
    wid = lax.axis_index("s") * NC + lax.axis_index("c")
    pix0 = wid * PIX_PER_W

    def chunk(g, _):
        base = pix0 + g * K
        pltpu.sync_copy(idx2.at[:, pl.ds(base, K)], idx_v)
        pltpu.sync_copy(w4.at[:, pl.ds(base, K)], w_v)
        if not _SKIP_GATHER:
            descs = [
                pltpu.async_copy(table.at[idx_v.at[i]], rows_v.at[i], sem)
                for i in range(2)
            ]
            for d in descs:
                d.wait()

        def xg_body(xg, _):
            ci = lax.broadcasted_iota(jnp.int32, (L,), 0)
            zz = jnp.full((L,), 0, jnp.int32)
            w16 = [w_v[i, pl.ds(xg * L, L)] for i in range(4)]
            pbase = xg * L
            for j in range(L):
                wj = [_lane_bcast(w16[i], j) for i in range(4)]
                p = pbase + j
                pp = zz + p
                us = [[rows_v[i, p, pl.ds(s * 64 + cb * L, L)]
                       for cb in range(3)]
                      for i in range(2) for s in range(2)]
                for cb in range(3):
                    e0, o0 = _halves(us[0][cb])
                    e1, o1 = _halves(us[1][cb])
                    e2, o2 = _halves(us[2][cb])
                    e3, o3 = _halves(us[3][cb])
                    acc_e = e0 * wj[0] + e1 * wj[1] + e2 * wj[2] + e3 * wj[3]
                    acc_o = o0 * wj[0] + o1 * wj[1] + o2 * wj[2] + o3 * wj[3]
                    plsc.store_scatter(ob_v, [ci + cb * L, pp], acc_e)
                    plsc.store_scatter(ob_v, [ci + (C // 2 + cb * L), pp],
                                       acc_o)
            return 0

        if not _SKIP_COMPUTE:
            lax.fori_loop(0, K // L, xg_body, 0)

        bb = base // HW
        rem = base - bb * HW
        yy = rem // W
        xx = rem - yy * W
        pltpu.sync_copy(ob_v.at[:, pl.ds(0, K)], out.at[bb, :, yy, pl.ds(xx, K)])
        return 0

    lax.fori_loop(0, CHUNKS, chunk, 0)


@functools.lru_cache(maxsize=1)
def _sc_warp():
    return pl.kernel(
        _sc_warp_body,
        out_type=jax.ShapeDtypeStruct((B, C, H, W), jnp.float32),
        mesh=plsc.VectorSubcoreMesh(core_axis_name="c", subcore_axis_name="s"),
        compiler_params=pltpu.CompilerParams(needs_layout_passes=False, use_tc_tiling_on_sc=True),
        scratch_types=[
            pltpu.VMEM((2, K), jnp.int32),
            pltpu.VMEM((4, K), jnp.float32),
            pltpu.VMEM((2, K, CP), jnp.int32),
            pltpu.VMEM((C, K + 1), jnp.float32),
            pltpu.SemaphoreType.DMA,
        ],
    )


def _build_table(input1):
    t = jnp.transpose(input1, (0, 2, 3, 1)).reshape(N, C)
    t = t.astype(jnp.bfloat16)
    t = jnp.stack([t[:, : C // 2], t[:, C // 2:]], axis=2)
    ti = lax.bitcast_convert_type(t, jnp.int32)
    ti = jnp.pad(ti, ((0, 0), (0, CP // 2 - C // 2)))
    nxt = jnp.roll(ti, -1, axis=0)
    return jnp.concatenate([ti, nxt], axis=1)


def kernel(input1, input2):
    if input2.shape[1] == 3:
        input2 = input2[:, :2, :, :]
    table = _build_table(input1)
    idx2, w4 = _prep(input2)
    idx2 = idx2.reshape(2, N)
    w4 = w4.reshape(4, N)
    return _sc_warp()(table, idx2, w4)


# speedup vs baseline: 1.6599x; 1.6597x over previous
import functools

import jax
import jax.numpy as jnp
from jax import lax
from jax.experimental import pallas as pl
from jax.experimental.pallas import tpu as pltpu
from jax.experimental.pallas import tpu_sc as plsc

B, C, H, W = 2, 96, 512, 512
HW = H * W
N = B * HW
CP = 128

NC, NS, L = 2, 16, 16
NW = NC * NS
PIX_PER_W = N // NW
K = 128
CHUNKS = PIX_PER_W // K

_HB = 128
_SKIP_COMPUTE = False
_SKIP_GATHER = False

_BCAST_DNUMS = lax.GatherDimensionNumbers(
    offset_dims=(), collapsed_slice_dims=(0,), start_index_map=(0,))


def _lane_bcast(vec, j):
    idx = jnp.full((L, 1), j, jnp.int32)
    return lax.gather(vec, idx, _BCAST_DNUMS, (1,),
                      mode=lax.GatherScatterMode.PROMISE_IN_BOUNDS)


def _prep_body(in_ref, idx_ref, w_ref):
    b = pl.program_id(0)
    h = pl.program_id(1)
    d = in_ref[0]
    dx = d[0]
    dy = d[1]
    gy = lax.broadcasted_iota(jnp.int32, (_HB, W), 0).astype(jnp.float32) + (
        h * _HB).astype(jnp.float32)
    gx = lax.broadcasted_iota(jnp.int32, (_HB, W), 1).astype(jnp.float32)
    xf = gx + dx
    yf = gy + dy
    x0 = jnp.floor(xf)
    y0 = jnp.floor(yf)
    y1 = y0 + 1.0
    wx1 = xf - x0
    wx0 = 1.0 - wx1
    wy1 = yf - y0
    wy0 = 1.0 - wy1
    fW = jnp.float32(W - 1)
    fH = jnp.float32(H - 1)
    xbf = jnp.clip(x0, 0.0, jnp.float32(W - 2))
    eqA = (x0 == xbf).astype(jnp.float32)
    wA = wx0 * eqA + wx1 * (x0 == -1.0).astype(jnp.float32)
    wB = wx1 * eqA + wx0 * (x0 == fW).astype(jnp.float32)
    vy0 = ((y0 >= 0) & (y0 <= fH)).astype(jnp.float32)
    vy1 = ((y1 >= 0) & (y1 <= fH)).astype(jnp.float32)
    xb = xbf.astype(jnp.int32)
    y0c = jnp.clip(y0, 0.0, fH).astype(jnp.int32)
    y1c = jnp.clip(y1, 0.0, fH).astype(jnp.int32)
    base = b * HW
    idx_ref[0, 0] = base + y0c * W + xb
    idx_ref[1, 0] = base + y1c * W + xb
    wT = wy0 * vy0
    wBo = wy1 * vy1
    w_ref[0, 0] = wA * wT
    w_ref[1, 0] = wB * wT
    w_ref[2, 0] = wA * wBo
    w_ref[3, 0] = wB * wBo


def _prep(input2, interpret=False):
    return pl.pallas_call(
        _prep_body,
        grid=(B, H // _HB),
        in_specs=[pl.BlockSpec((1, 2, _HB, W), lambda b, h: (b, 0, h, 0))],
        out_specs=[
            pl.BlockSpec((2, 1, _HB, W), lambda b, h: (0, b, h, 0)),
            pl.BlockSpec((4, 1, _HB, W), lambda b, h: (0, b, h, 0)),
        ],
        out_shape=[
            jax.ShapeDtypeStruct((2, B, H, W), jnp.int32),
            jax.ShapeDtypeStruct((4, B, H, W), jnp.float32),
        ],
        interpret=interpret,
    )(input2)


_M0 = -65536


def _halves(u32):
    ev = plsc.bitcast(u32 << 16, jnp.float32)
    od = plsc.bitcast(u32 & _M0, jnp.float32)
    return ev, od


def _sc_warp_body(table, idx2, w4, out, idx_v, w_v, rows_v, ob_v, sem):
    wid = lax.axis_index("s") * NC + lax.axis_index("c")
    pix0 = wid * PIX_PER_W

    def chunk(g, _):
        base = pix0 + g * K
        pltpu.sync_copy(idx2.at[:, pl.ds(base, K)], idx_v)
        pltpu.sync_copy(w4.at[:, pl.ds(base, K)], w_v)
        if not _SKIP_GATHER:
            descs = [
                pltpu.async_copy(table.at[idx_v.at[i]], rows_v.at[i], sem)
                for i in range(2)
            ]
            for d in descs:
                d.wait()

        def xg_body(xg, _):
            ci = lax.broadcasted_iota(jnp.int32, (L,), 0)
            zz = jnp.full((L,), 0, jnp.int32)
            w16 = [w_v[i, pl.ds(xg * L, L)] for i in range(4)]
            pbase = xg * L
            for j in range(L):
                wj = [_lane_bcast(w16[i], j) for i in range(4)]
                p = pbase + j
                pp = zz + p
                us = [[rows_v[i, p, pl.ds(s * 64 + cb * L, L)]
                       for cb in range(3)]
                      for i in range(2) for s in range(2)]
                for cb in range(3):
                    e0, o0 = _halves(us[0][cb])
                    e1, o1 = _halves(us[1][cb])
                    e2, o2 = _halves(us[2][cb])
                    e3, o3 = _halves(us[3][cb])
                    acc_e = e0 * wj[0] + e1 * wj[1] + e2 * wj[2] + e3 * wj[3]
                    acc_o = o0 * wj[0] + o1 * wj[1] + o2 * wj[2] + o3 * wj[3]
                    plsc.store_scatter(ob_v, [ci + cb * L, pp], acc_e)
                    plsc.store_scatter(ob_v, [ci + (C // 2 + cb * L), pp],
                                       acc_o)
            return 0

        if not _SKIP_COMPUTE:
            lax.fori_loop(0, K // L, xg_body, 0)

        bb = base // HW
        rem = base - bb * HW
        yy = rem // W
        xx = rem - yy * W
        pltpu.sync_copy(ob_v.at[:, pl.ds(0, K)], out.at[bb, :, yy, pl.ds(xx, K)])
        return 0

    lax.fori_loop(0, CHUNKS, chunk, 0)


@functools.lru_cache(maxsize=1)
def _sc_warp():
    return pl.kernel(
        _sc_warp_body,
        out_type=jax.ShapeDtypeStruct((B, C, H, W), jnp.float32),
        mesh=plsc.VectorSubcoreMesh(core_axis_name="c", subcore_axis_name="s"),
        compiler_params=pltpu.CompilerParams(needs_layout_passes=False),
        scratch_types=[
            pltpu.VMEM((2, K), jnp.int32),
            pltpu.VMEM((4, K), jnp.float32),
            pltpu.VMEM((2, K, CP), jnp.int32),
            pltpu.VMEM((C, K + 1), jnp.float32),
            pltpu.SemaphoreType.DMA,
        ],
    )


_HB2 = 8


def _table_body(in_ref, out_ref):
    M = _HB2 * W
    x = in_ref[0].reshape(C, M)
    u = lax.bitcast_convert_type(x.astype(jnp.bfloat16), jnp.uint16)
    lo = u[: C // 2].astype(jnp.uint32)
    hi = u[C // 2:].astype(jnp.uint32)
    w = lo | (hi << 16)
    wn = jnp.concatenate([w[:, 1:], w[:, :1]], axis=1)
    z = jnp.zeros((CP // 2 - C // 2, M), jnp.uint32)
    big = jnp.concatenate([w, z, wn, z], axis=0)
    out_ref[...] = lax.bitcast_convert_type(jnp.transpose(big), jnp.int32)


def _build_table(input1):
    return pl.pallas_call(
        _table_body,
        grid=(B, H // _HB2),
        in_specs=[pl.BlockSpec((1, C, _HB2, W), lambda b, h: (b, 0, h, 0))],
        out_specs=[pl.BlockSpec((_HB2 * W, CP),
                                lambda b, h: (b * (H // _HB2) + h, 0))],
        out_shape=[jax.ShapeDtypeStruct((N, CP), jnp.int32)],
    )(input1)[0]


def kernel(input1, input2):
    if input2.shape[1] == 3:
        input2 = input2[:, :2, :, :]
    table = _build_table(input1)
    idx2, w4 = _prep(input2)
    idx2 = idx2.reshape(2, N)
    w4 = w4.reshape(4, N)
    return _sc_warp()(table, idx2, w4)


# speedup vs baseline: 2.0306x; 1.2233x over previous
import functools

import jax
import jax.numpy as jnp
from jax import lax
from jax.experimental import pallas as pl
from jax.experimental.pallas import tpu as pltpu
from jax.experimental.pallas import tpu_sc as plsc

B, C, H, W = 2, 96, 512, 512
HW = H * W
N = B * HW
CP = 128

NC, NS, L = 2, 16, 16
NW = NC * NS
PIX_PER_W = N // NW
K = 128
CHUNKS = PIX_PER_W // K

_HB = 128
_SKIP_COMPUTE = False
_SKIP_GATHER = False

_BCAST_DNUMS = lax.GatherDimensionNumbers(
    offset_dims=(), collapsed_slice_dims=(0,), start_index_map=(0,))


def _lane_bcast(vec, j):
    idx = jnp.full((L, 1), j, jnp.int32)
    return lax.gather(vec, idx, _BCAST_DNUMS, (1,),
                      mode=lax.GatherScatterMode.PROMISE_IN_BOUNDS)


def _prep_body(in_ref, idx_ref):
    b = pl.program_id(0)
    h = pl.program_id(1)
    d = in_ref[0]
    dx = d[0]
    dy = d[1]
    gy = lax.broadcasted_iota(jnp.int32, (_HB, W), 0).astype(jnp.float32) + (
        h * _HB).astype(jnp.float32)
    gx = lax.broadcasted_iota(jnp.int32, (_HB, W), 1).astype(jnp.float32)
    xf = gx + dx
    yf = gy + dy
    x0 = jnp.floor(xf)
    y0 = jnp.floor(yf)
    y1 = y0 + 1.0
    wx1 = xf - x0
    wx0 = 1.0 - wx1
    wy1 = yf - y0
    wy0 = 1.0 - wy1
    fW = jnp.float32(W - 1)
    fH = jnp.float32(H - 1)
    xbf = jnp.clip(x0, 0.0, jnp.float32(W - 2))
    eqA = (x0 == xbf).astype(jnp.float32)
    wA = wx0 * eqA + wx1 * (x0 == -1.0).astype(jnp.float32)
    wB = wx1 * eqA + wx0 * (x0 == fW).astype(jnp.float32)
    vy0 = ((y0 >= 0) & (y0 <= fH)).astype(jnp.float32)
    vy1 = ((y1 >= 0) & (y1 <= fH)).astype(jnp.float32)
    xb = xbf.astype(jnp.int32)
    y0c = jnp.clip(y0, 0.0, fH).astype(jnp.int32)
    y1c = jnp.clip(y1, 0.0, fH).astype(jnp.int32)
    base = b * HW
    idx_ref[0, 0] = base + y0c * W + xb
    idx_ref[1, 0] = base + y1c * W + xb
    wT = wy0 * vy0
    wBo = wy1 * vy1
    bc = lambda v: lax.bitcast_convert_type(v, jnp.int32)
    idx_ref[2, 0] = bc(wA * wT)
    idx_ref[3, 0] = bc(wB * wT)
    idx_ref[4, 0] = bc(wA * wBo)
    idx_ref[5, 0] = bc(wB * wBo)


def _prep(input2, interpret=False):
    return pl.pallas_call(
        _prep_body,
        grid=(B, H // _HB),
        in_specs=[pl.BlockSpec((1, 2, _HB, W), lambda b, h: (b, 0, h, 0))],
        out_specs=[
            pl.BlockSpec((6, 1, _HB, W), lambda b, h: (0, b, h, 0)),
        ],
        out_shape=[
            jax.ShapeDtypeStruct((6, B, H, W), jnp.int32),
        ],
        interpret=interpret,
    )(input2)


_M0 = -65536


def _halves(u32):
    ev = plsc.bitcast(u32 << 16, jnp.float32)
    od = plsc.bitcast(u32 & _M0, jnp.float32)
    return ev, od


def _sc_warp_body(table, iw, out, iw_v0, iw_v1, rows_v0, rows_v1, ob_v0,
                  ob_v1, sem_iw, sem_g, sem_out):
    iw_b = (iw_v0, iw_v1)
    rows_b = (rows_v0, rows_v1)
    ob_b = (ob_v0, ob_v1)
    wid = lax.axis_index("s") * NC + lax.axis_index("c")
    pix0 = wid * PIX_PER_W

    def iw_start(b3, g):
        pltpu.async_copy(iw.at[:, pl.ds(pix0 + g * K, K)], iw_b[b3], sem_iw)

    def iw_wait(b3, g):
        pltpu.make_async_copy(
            iw.at[:, pl.ds(pix0 + g * K, K)], iw_b[b3], sem_iw).wait()

    def gather_start(b2, b3):
        for i in range(2):
            pltpu.async_copy(table.at[iw_b[b3].at[i]], rows_b[b2].at[i], sem_g)

    def gather_wait(b2, b3):
        for i in range(2):
            pltpu.make_async_copy(
                table.at[iw_b[b3].at[i]], rows_b[b2].at[i], sem_g).wait()

    def out_dst(g):
        base = pix0 + g * K
        bb = base // HW
        rem = base - bb * HW
        yy = rem // W
        xx = rem - yy * W
        return out.at[bb, :, yy, pl.ds(xx, K)]

    def out_start(b2, g):
        pltpu.async_copy(ob_b[b2].at[:, pl.ds(0, K)], out_dst(g), sem_out)

    def out_wait(b2, g):
        pltpu.make_async_copy(ob_b[b2].at[:, pl.ds(0, K)], out_dst(g),
                              sem_out).wait()

    def compute(b2, b3):
        def xg_body(xg, _):
            ci = lax.broadcasted_iota(jnp.int32, (L,), 0)
            zz = jnp.full((L,), 0, jnp.int32)
            w16 = [plsc.bitcast(iw_b[b3][2 + i, pl.ds(xg * L, L)],
                                jnp.float32)
                   for i in range(4)]
            pbase = xg * L
            for j in range(L):
                wj = [_lane_bcast(w16[i], j) for i in range(4)]
                p = pbase + j
                pp = zz + p
                us = [[rows_b[b2][i, p, pl.ds(s * 64 + cb * L, L)]
                       for cb in range(3)]
                      for i in range(2) for s in range(2)]
                for cb in range(3):
                    e0, o0 = _halves(us[0][cb])
                    e1, o1 = _halves(us[1][cb])
                    e2, o2 = _halves(us[2][cb])
                    e3, o3 = _halves(us[3][cb])
                    acc_e = e0 * wj[0] + e1 * wj[1] + e2 * wj[2] + e3 * wj[3]
                    acc_o = o0 * wj[0] + o1 * wj[1] + o2 * wj[2] + o3 * wj[3]
                    plsc.store_scatter(ob_b[b2], [ci + cb * L, pp], acc_e)
                    plsc.store_scatter(ob_b[b2],
                                       [ci + (C // 2 + cb * L), pp], acc_o)
            return 0

        lax.fori_loop(0, K // L, xg_body, 0)

    pltpu.sync_copy(iw.at[:, pl.ds(pix0, K)], iw_v0)
    gather_start(0, 0)
    iw_start(1, 1)

    def pair(gi, _):
        for ph in range(2):
            g = gi * 2 + ph
            b2 = ph
            gather_wait(b2, b2)

            @pl.when(g + 1 < CHUNKS)
            def _():
                iw_wait(1 - b2, g + 1)
                gather_start(1 - b2, 1 - b2)

            @pl.when(g >= 2)
            def _():
                out_wait(b2, g - 2)

            compute(b2, b2)

            @pl.when(g + 2 < CHUNKS)
            def _():
                iw_start(b2, g + 2)

            out_start(b2, g)
        return 0

    lax.fori_loop(0, CHUNKS // 2, pair, 0)
    out_wait(0, CHUNKS - 2)
    out_wait(1, CHUNKS - 1)


@functools.lru_cache(maxsize=1)
def _sc_warp():
    return pl.kernel(
        _sc_warp_body,
        out_type=jax.ShapeDtypeStruct((B, C, H, W), jnp.float32),
        mesh=plsc.VectorSubcoreMesh(core_axis_name="c", subcore_axis_name="s"),
        compiler_params=pltpu.CompilerParams(needs_layout_passes=False),
        scratch_types=[
            pltpu.VMEM((6, K), jnp.int32),
            pltpu.VMEM((6, K), jnp.int32),
            pltpu.VMEM((2, K, CP), jnp.int32),
            pltpu.VMEM((2, K, CP), jnp.int32),
            pltpu.VMEM((C, K + 1), jnp.float32),
            pltpu.VMEM((C, K + 1), jnp.float32),
            pltpu.SemaphoreType.DMA,
            pltpu.SemaphoreType.DMA,
            pltpu.SemaphoreType.DMA,
        ],
    )


_HB2 = 8


def _table_body(in_ref, out_ref):
    M = _HB2 * W
    x = in_ref[0].reshape(C, M)
    u = lax.bitcast_convert_type(x.astype(jnp.bfloat16), jnp.uint16)
    lo = u[: C // 2].astype(jnp.uint32)
    hi = u[C // 2:].astype(jnp.uint32)
    w = lo | (hi << 16)
    wn = jnp.concatenate([w[:, 1:], w[:, :1]], axis=1)
    z = jnp.zeros((CP // 2 - C // 2, M), jnp.uint32)
    big = jnp.concatenate([w, z, wn, z], axis=0)
    out_ref[...] = lax.bitcast_convert_type(jnp.transpose(big), jnp.int32)


def _build_table(input1):
    return pl.pallas_call(
        _table_body,
        grid=(B, H // _HB2),
        in_specs=[pl.BlockSpec((1, C, _HB2, W), lambda b, h: (b, 0, h, 0))],
        out_specs=[pl.BlockSpec((_HB2 * W, CP),
                                lambda b, h: (b * (H // _HB2) + h, 0))],
        out_shape=[jax.ShapeDtypeStruct((N, CP), jnp.int32)],
    )(input1)[0]


def kernel(input1, input2):
    if input2.shape[1] == 3:
        input2 = input2[:, :2, :, :]
    table = _build_table(input1)
    iw = _prep(input2)[0].reshape(6, N)
    return _sc_warp()(table, iw)


# speedup vs baseline: 2.1317x; 1.0498x over previous
import functools

import jax
import jax.numpy as jnp
from jax import lax
from jax.experimental import pallas as pl
from jax.experimental.pallas import tpu as pltpu
from jax.experimental.pallas import tpu_sc as plsc

B, C, H, W = 2, 96, 512, 512
HW = H * W
N = B * HW
CP = 128

NC, NS, L = 2, 16, 16
NW = NC * NS
PIX_PER_W = N // NW
K = 128
CHUNKS = PIX_PER_W // K

_HB = 128
_SKIP_COMPUTE = False
_SKIP_GATHER = False

_BCAST_DNUMS = lax.GatherDimensionNumbers(
    offset_dims=(), collapsed_slice_dims=(0,), start_index_map=(0,))


def _lane_bcast(vec, j):
    idx = jnp.full((L, 1), j, jnp.int32)
    return lax.gather(vec, idx, _BCAST_DNUMS, (1,),
                      mode=lax.GatherScatterMode.PROMISE_IN_BOUNDS)


def _prep_body(in_ref, idx_ref):
    b = pl.program_id(0)
    h = pl.program_id(1)
    d = in_ref[0]
    dx = d[0]
    dy = d[1]
    gy = lax.broadcasted_iota(jnp.int32, (_HB, W), 0).astype(jnp.float32) + (
        h * _HB).astype(jnp.float32)
    gx = lax.broadcasted_iota(jnp.int32, (_HB, W), 1).astype(jnp.float32)
    xf = gx + dx
    yf = gy + dy
    x0 = jnp.floor(xf)
    y0 = jnp.floor(yf)
    y1 = y0 + 1.0
    wx1 = xf - x0
    wx0 = 1.0 - wx1
    wy1 = yf - y0
    wy0 = 1.0 - wy1
    fW = jnp.float32(W - 1)
    fH = jnp.float32(H - 1)
    xbf = jnp.clip(x0, 0.0, jnp.float32(W - 2))
    eqA = (x0 == xbf).astype(jnp.float32)
    wA = wx0 * eqA + wx1 * (x0 == -1.0).astype(jnp.float32)
    wB = wx1 * eqA + wx0 * (x0 == fW).astype(jnp.float32)
    vy0 = ((y0 >= 0) & (y0 <= fH)).astype(jnp.float32)
    vy1 = ((y1 >= 0) & (y1 <= fH)).astype(jnp.float32)
    xb = xbf.astype(jnp.int32)
    y0c = jnp.clip(y0, 0.0, fH).astype(jnp.int32)
    y1c = jnp.clip(y1, 0.0, fH).astype(jnp.int32)
    base = b * HW
    idx_ref[0, 0] = base + y0c * W + xb
    idx_ref[1, 0] = base + y1c * W + xb
    wT = wy0 * vy0
    wBo = wy1 * vy1
    def bc(v):
        u = lax.bitcast_convert_type(v.astype(jnp.bfloat16),
                                     jnp.uint16).astype(jnp.uint32)
        return lax.bitcast_convert_type(u | (u << 16), jnp.int32)

    idx_ref[2, 0] = bc(wA * wT)
    idx_ref[3, 0] = bc(wB * wT)
    idx_ref[4, 0] = bc(wA * wBo)
    idx_ref[5, 0] = bc(wB * wBo)


def _prep(input2, interpret=False):
    return pl.pallas_call(
        _prep_body,
        grid=(B, H // _HB),
        in_specs=[pl.BlockSpec((1, 2, _HB, W), lambda b, h: (b, 0, h, 0))],
        out_specs=[
            pl.BlockSpec((6, 1, _HB, W), lambda b, h: (0, b, h, 0)),
        ],
        out_shape=[
            jax.ShapeDtypeStruct((6, B, H, W), jnp.int32),
        ],
        interpret=interpret,
    )(input2)


_M0 = -65536


def _halves(u32):
    ev = plsc.bitcast(u32 << 16, jnp.float32)
    od = plsc.bitcast(u32 & _M0, jnp.float32)
    return ev, od


def _sc_warp_body(table, iw, out, iw_v0, iw_v1, rows_v0, rows_v1, ob_v0,
                  ob_v1, sem_iw, sem_g, sem_out):
    iw_b = (iw_v0, iw_v1)
    rows_b = (rows_v0, rows_v1)
    ob_b = (ob_v0, ob_v1)
    wid = lax.axis_index("s") * NC + lax.axis_index("c")
    pix0 = wid * PIX_PER_W

    def iw_start(b3, g):
        pltpu.async_copy(iw.at[:, pl.ds(pix0 + g * K, K)], iw_b[b3], sem_iw)

    def iw_wait(b3, g):
        pltpu.make_async_copy(
            iw.at[:, pl.ds(pix0 + g * K, K)], iw_b[b3], sem_iw).wait()

    def gather_start(b2, b3):
        for i in range(2):
            pltpu.async_copy(table.at[iw_b[b3].at[i]], rows_b[b2].at[i], sem_g)

    def gather_wait(b2, b3):
        for i in range(2):
            pltpu.make_async_copy(
                table.at[iw_b[b3].at[i]], rows_b[b2].at[i], sem_g).wait()

    def out_dst(g):
        base = pix0 + g * K
        bb = base // HW
        rem = base - bb * HW
        yy = rem // W
        xx = rem - yy * W
        return out.at[bb, :, yy, pl.ds(xx, K)]

    def out_start(b2, g):
        pltpu.async_copy(ob_b[b2].at[:, pl.ds(0, K)], out_dst(g), sem_out)

    def out_wait(b2, g):
        pltpu.make_async_copy(ob_b[b2].at[:, pl.ds(0, K)], out_dst(g),
                              sem_out).wait()

    def compute(b2, b3):
        def xg_body(xg, _):
            ci = lax.broadcasted_iota(jnp.int32, (L,), 0)
            zz = jnp.full((L,), 0, jnp.int32)
            w16 = [iw_b[b3][2 + i, pl.ds(xg * L, L)] for i in range(4)]
            pbase = xg * L
            for j in range(L):
                wj = [plsc.bitcast(_lane_bcast(w16[i], j), jnp.bfloat16)
                      for i in range(4)]
                p = pbase + j
                pp = zz + p
                us = [[plsc.bitcast(rows_b[b2][i, p,
                                               pl.ds(s * 64 + cb * L, L)],
                                    jnp.bfloat16)
                       for cb in range(3)]
                      for i in range(2) for s in range(2)]
                for cb in range(3):
                    acc = (us[0][cb] * wj[0] + us[1][cb] * wj[1]
                           + us[2][cb] * wj[2] + us[3][cb] * wj[3])
                    acc_e, acc_o = _halves(plsc.bitcast(acc, jnp.int32))
                    plsc.store_scatter(ob_b[b2], [ci + cb * L, pp], acc_e)
                    plsc.store_scatter(ob_b[b2],
                                       [ci + (C // 2 + cb * L), pp], acc_o)
            return 0

        lax.fori_loop(0, K // L, xg_body, 0)

    pltpu.sync_copy(iw.at[:, pl.ds(pix0, K)], iw_v0)
    gather_start(0, 0)
    iw_start(1, 1)

    def pair(gi, _):
        for ph in range(2):
            g = gi * 2 + ph
            b2 = ph
            gather_wait(b2, b2)

            @pl.when(g + 1 < CHUNKS)
            def _():
                iw_wait(1 - b2, g + 1)
                gather_start(1 - b2, 1 - b2)

            @pl.when(g >= 2)
            def _():
                out_wait(b2, g - 2)

            if not _SKIP_COMPUTE:
                compute(b2, b2)

            @pl.when(g + 2 < CHUNKS)
            def _():
                iw_start(b2, g + 2)

            out_start(b2, g)
        return 0

    lax.fori_loop(0, CHUNKS // 2, pair, 0)
    out_wait(0, CHUNKS - 2)
    out_wait(1, CHUNKS - 1)


@functools.lru_cache(maxsize=1)
def _sc_warp():
    return pl.kernel(
        _sc_warp_body,
        out_type=jax.ShapeDtypeStruct((B, C, H, W), jnp.float32),
        mesh=plsc.VectorSubcoreMesh(core_axis_name="c", subcore_axis_name="s"),
        compiler_params=pltpu.CompilerParams(needs_layout_passes=False),
        scratch_types=[
            pltpu.VMEM((6, K), jnp.int32),
            pltpu.VMEM((6, K), jnp.int32),
            pltpu.VMEM((2, K, CP), jnp.int32),
            pltpu.VMEM((2, K, CP), jnp.int32),
            pltpu.VMEM((C, K + 1), jnp.float32),
            pltpu.VMEM((C, K + 1), jnp.float32),
            pltpu.SemaphoreType.DMA,
            pltpu.SemaphoreType.DMA,
            pltpu.SemaphoreType.DMA,
        ],
    )


_HB2 = 8


def _table_body(in_ref, out_ref):
    M = _HB2 * W
    x = in_ref[0].reshape(C, M)
    u = lax.bitcast_convert_type(x.astype(jnp.bfloat16), jnp.uint16)
    lo = u[: C // 2].astype(jnp.uint32)
    hi = u[C // 2:].astype(jnp.uint32)
    w = lo | (hi << 16)
    wn = jnp.concatenate([w[:, 1:], w[:, :1]], axis=1)
    z = jnp.zeros((CP // 2 - C // 2, M), jnp.uint32)
    big = jnp.concatenate([w, z, wn, z], axis=0)
    out_ref[...] = lax.bitcast_convert_type(jnp.transpose(big), jnp.int32)


def _build_table(input1):
    return pl.pallas_call(
        _table_body,
        grid=(B, H // _HB2),
        in_specs=[pl.BlockSpec((1, C, _HB2, W), lambda b, h: (b, 0, h, 0))],
        out_specs=[pl.BlockSpec((_HB2 * W, CP),
                                lambda b, h: (b * (H // _HB2) + h, 0))],
        out_shape=[jax.ShapeDtypeStruct((N, CP), jnp.int32)],
    )(input1)[0]


def kernel(input1, input2):
    if input2.shape[1] == 3:
        input2 = input2[:, :2, :, :]
    table = _build_table(input1)
    iw = _prep(input2)[0].reshape(6, N)
    return _sc_warp()(table, iw)
